# baseline (speedup 1.0000x reference)
import functools

import jax
import jax.numpy as jnp
from jax import lax
from jax.experimental import pallas as pl
from jax.experimental.pallas import tpu as pltpu
from jax.experimental.pallas import tpu_sc as plsc

B = 2048
D = 768
E = 8
P = 2 * B
BT = 128
NW = 32
GPW = P // NW
CPW = B // NW
LANES = 16


def _router_body(x_ref, rw_ref, rb_ref, i1_ref, i2_ref, w1_ref, w2_ref):
    x = x_ref[...]
    logits = jnp.dot(x, rw_ref[...], preferred_element_type=jnp.float32)
    logits = logits + rb_ref[...][None, :]
    idx = lax.broadcasted_iota(jnp.int32, (B, E), 1)
    neg = jnp.float32(-1.7e38)
    v1 = jnp.max(logits, axis=1, keepdims=True)
    i1 = jnp.min(jnp.where(logits == v1, idx, E), axis=1, keepdims=True)
    masked = jnp.where(idx == i1, neg, logits)
    v2 = jnp.max(masked, axis=1, keepdims=True)
    i2 = jnp.min(jnp.where(masked == v2, idx, E), axis=1, keepdims=True)
    w1 = 1.0 / (1.0 + jnp.exp(v2 - v1))
    i1_ref[...] = i1
    i2_ref[...] = i2
    w1_ref[...] = w1
    w2_ref[...] = 1.0 - w1


def _router(x, root_w, root_b):
    return pl.pallas_call(
        _router_body,
        out_shape=(
            jax.ShapeDtypeStruct((B, 1), jnp.int32),
            jax.ShapeDtypeStruct((B, 1), jnp.int32),
            jax.ShapeDtypeStruct((B, 1), jnp.float32),
            jax.ShapeDtypeStruct((B, 1), jnp.float32),
        ),
    )(x, root_w, root_b)


def _gmm_body(xs_ref, se_ref, sw_ref, w_ref, b_ref, out_ref):
    se = se_ref[...]
    lo = jnp.min(se)
    hi = jnp.max(se)
    xs = xs_ref[...]
    out_ref[...] = jnp.zeros((BT, D), jnp.float32)
    for e in range(E):

        @pl.when(jnp.logical_and(lo <= e, e <= hi))
        def _():
            y = jnp.dot(xs, w_ref[e], preferred_element_type=jnp.float32)
            y = y + b_ref[e][None, :]
            m = (se == e).astype(jnp.float32)
            out_ref[...] += m * y

    out_ref[...] *= sw_ref[...]


def _gmm(xs, se, sw, sons_w, sons_b):
    return pl.pallas_call(
        _gmm_body,
        grid=(P // BT,),
        in_specs=[
            pl.BlockSpec((BT, D), lambda i: (i, 0)),
            pl.BlockSpec((BT, 1), lambda i: (i, 0)),
            pl.BlockSpec((BT, 1), lambda i: (i, 0)),
            pl.BlockSpec((E, D, D), lambda i: (0, 0, 0)),
            pl.BlockSpec((E, D), lambda i: (0, 0)),
        ],
        out_specs=pl.BlockSpec((BT, D), lambda i: (i, 0)),
        out_shape=jax.ShapeDtypeStruct((P, D), jnp.float32),
    )(xs, se, sw, sons_w, sons_b)


def _mesh():
    return plsc.VectorSubcoreMesh(
        core_axis_name="c", subcore_axis_name="s", num_cores=2, num_subcores=16
    )


@functools.cache
def _build_dispatch():
    @functools.partial(
        pl.kernel,
        out_type=jax.ShapeDtypeStruct((P, D), jnp.float32),
        mesh=_mesh(),
        scratch_types=[
            pltpu.VMEM((GPW,), jnp.int32),
            pltpu.VMEM((GPW, D), jnp.float32),
            pltpu.SemaphoreType.DMA,
        ],
    )
    def _dispatch_k(x_hbm, tokp_hbm, xs_hbm, idx_v, rows_v, sem):
        wid = lax.axis_index("s") * 2 + lax.axis_index("c")
        base = wid * GPW
        pltpu.sync_copy(tokp_hbm.at[pl.ds(base, GPW)], idx_v)
        pltpu.async_copy(x_hbm.at[idx_v], rows_v, sem).wait()
        pltpu.sync_copy(rows_v, xs_hbm.at[pl.ds(base, GPW)])

    return _dispatch_k


def _dispatch(x, tokp):
    return _build_dispatch()(x, tokp)


@functools.cache
def _build_combine():
    @functools.partial(
        pl.kernel,
        out_type=jax.ShapeDtypeStruct((B, D), jnp.float32),
        mesh=_mesh(),
        scratch_types=[
            pltpu.VMEM((CPW,), jnp.int32),
            pltpu.VMEM((CPW,), jnp.int32),
            pltpu.VMEM((CPW, D), jnp.float32),
            pltpu.VMEM((CPW, D), jnp.float32),
            pltpu.SemaphoreType.DMA,
        ],
    )
    def _combine_k(ys_hbm, q0_hbm, q1_hbm, out_hbm, q0_v, q1_v, b0, b1, sem):
        wid = lax.axis_index("s") * 2 + lax.axis_index("c")
        base = wid * CPW
        pltpu.sync_copy(q0_hbm.at[pl.ds(base, CPW)], q0_v)
        pltpu.sync_copy(q1_hbm.at[pl.ds(base, CPW)], q1_v)
        cp0 = pltpu.async_copy(ys_hbm.at[q0_v], b0, sem)
        cp1 = pltpu.async_copy(ys_hbm.at[q1_v], b1, sem)
        cp0.wait()
        cp1.wait()

        def _row(r, carry):
            def _col(c, carry2):
                sl = pl.ds(c * LANES, LANES)
                b0[r, sl] = b0[r, sl] + b1[r, sl]
                return carry2

            return lax.fori_loop(0, D // LANES, _col, carry, unroll=8)

        lax.fori_loop(0, CPW, _row, 0)
        pltpu.sync_copy(b0, out_hbm.at[pl.ds(base, CPW)])

    return _combine_k


def _combine(ys, q0, q1):
    return _build_combine()(ys, q0, q1)


@jax.jit
def kernel(x, root_w, root_b, sons_w, sons_b):
    i1, i2, w1, w2 = _router(x, root_w, root_b)
    keys = jnp.concatenate([i1[:, 0], i2[:, 0]])
    wflat = jnp.concatenate([w1[:, 0], w2[:, 0]])
    perm = jnp.argsort(keys)
    tokp = (perm & (B - 1)).astype(jnp.int32)
    sorted_e = keys[perm]
    sorted_w = wflat[perm]
    dest = (
        jnp.zeros((P,), jnp.int32)
        .at[perm]
        .set(jnp.arange(P, dtype=jnp.int32))
    )
    q0, q1 = dest[:B], dest[B:]

    xs = _dispatch(x, tokp)
    ys = _gmm(xs, sorted_e.reshape(P, 1), sorted_w.reshape(P, 1), sons_w, sons_b)
    out = _combine(ys, q0, q1)
    return out[:, None, :]


# speedup vs baseline: 2.4091x; 2.4091x over previous
import functools

import jax
import jax.numpy as jnp
from jax.experimental import pallas as pl

B = 2048
D = 768
E = 8
TOP_K = 2
BT = 512


def _fused_moe_kernel(x_ref, rw_ref, rb_ref, sw_ref, sb_ref, out_ref):
    x = x_ref[...]
    logits = jnp.dot(x, rw_ref[...], preferred_element_type=jnp.float32)
    logits = logits + rb_ref[...][None, :]

    idx = jax.lax.broadcasted_iota(jnp.int32, (BT, E), 1)
    neg = jnp.float32(-1.7e38)

    v1 = jnp.max(logits, axis=1, keepdims=True)
    i1 = jnp.min(jnp.where(logits == v1, idx, E), axis=1, keepdims=True)
    masked = jnp.where(idx == i1, neg, logits)
    v2 = jnp.max(masked, axis=1, keepdims=True)
    i2 = jnp.min(jnp.where(masked == v2, idx, E), axis=1, keepdims=True)

    w1 = 1.0 / (1.0 + jnp.exp(v2 - v1))
    w2 = 1.0 - w1
    wmat = w1 * (idx == i1).astype(jnp.float32) + w2 * (idx == i2).astype(
        jnp.float32
    )

    acc = jnp.dot(wmat, sb_ref[...], preferred_element_type=jnp.float32)
    for e in range(E):
        y = jnp.dot(x, sw_ref[e], preferred_element_type=jnp.float32)
        acc = acc + wmat[:, e : e + 1] * y
    out_ref[...] = acc


@jax.jit
def kernel(x, root_w, root_b, sons_w, sons_b):
    out = pl.pallas_call(
        _fused_moe_kernel,
        grid=(B // BT,),
        in_specs=[
            pl.BlockSpec((BT, D), lambda i: (i, 0)),
            pl.BlockSpec((D, E), lambda i: (0, 0)),
            pl.BlockSpec((E,), lambda i: (0,)),
            pl.BlockSpec((E, D, D), lambda i: (0, 0, 0)),
            pl.BlockSpec((E, D), lambda i: (0, 0)),
        ],
        out_specs=pl.BlockSpec((BT, D), lambda i: (i, 0)),
        out_shape=jax.ShapeDtypeStruct((B, D), jnp.float32),
    )(x, root_w, root_b, sons_w, sons_b)
    return out[:, None, :]


# speedup vs baseline: 2.4289x; 1.0082x over previous
import functools

import jax
import jax.numpy as jnp
from jax.experimental import pallas as pl

B = 2048
D = 768
E = 8
TOP_K = 2
BT = 1024


def _fused_moe_kernel(x_ref, rw_ref, rb_ref, sw_ref, sb_ref, out_ref):
    x = x_ref[...]
    logits = jnp.dot(x, rw_ref[...], preferred_element_type=jnp.float32)
    logits = logits + rb_ref[...][None, :]

    idx = jax.lax.broadcasted_iota(jnp.int32, (BT, E), 1)
    neg = jnp.float32(-1.7e38)

    v1 = jnp.max(logits, axis=1, keepdims=True)
    i1 = jnp.min(jnp.where(logits == v1, idx, E), axis=1, keepdims=True)
    masked = jnp.where(idx == i1, neg, logits)
    v2 = jnp.max(masked, axis=1, keepdims=True)
    i2 = jnp.min(jnp.where(masked == v2, idx, E), axis=1, keepdims=True)

    w1 = 1.0 / (1.0 + jnp.exp(v2 - v1))
    w2 = 1.0 - w1
    wmat = w1 * (idx == i1).astype(jnp.float32) + w2 * (idx == i2).astype(
        jnp.float32
    )

    acc = jnp.dot(wmat, sb_ref[...], preferred_element_type=jnp.float32)
    for e in range(E):
        y = jnp.dot(x, sw_ref[e], preferred_element_type=jnp.float32)
        acc = acc + wmat[:, e : e + 1] * y
    out_ref[...] = acc


@jax.jit
def kernel(x, root_w, root_b, sons_w, sons_b):
    out = pl.pallas_call(
        _fused_moe_kernel,
        grid=(B // BT,),
        in_specs=[
            pl.BlockSpec((BT, D), lambda i: (i, 0)),
            pl.BlockSpec((D, E), lambda i: (0, 0)),
            pl.BlockSpec((E,), lambda i: (0,)),
            pl.BlockSpec((E, D, D), lambda i: (0, 0, 0)),
            pl.BlockSpec((E, D), lambda i: (0, 0)),
        ],
        out_specs=pl.BlockSpec((BT, D), lambda i: (i, 0)),
        out_shape=jax.ShapeDtypeStruct((B, D), jnp.float32),
    )(x, root_w, root_b, sons_w, sons_b)
    return out[:, None, :]
